# baseline (speedup 1.0000x reference)
import functools

import jax
import jax.numpy as jnp
from jax.experimental import pallas as pl
from jax.experimental.pallas import tpu as pltpu

EMBED = 1024
NUM_HEADS = 16
HEAD_DIM = EMBED // NUM_HEADS
NUM_BLOCKS = 64
BS = 64
N_RAND = 3
SCALING = HEAD_DIM ** (-0.5)
NEG_INF = -1e30


def _proj_body(x_ref, w_ref, b_ref, o_ref):
    acc = jnp.dot(x_ref[...], w_ref[...], preferred_element_type=jnp.float32)
    o_ref[...] = acc + b_ref[...]


def _qkv_proj(x2d, w_all, b_all, block_m=512):
    m = x2d.shape[0]
    n = w_all.shape[1]
    k = x2d.shape[1]
    return pl.pallas_call(
        _proj_body,
        grid=(m // block_m,),
        in_specs=[
            pl.BlockSpec((block_m, k), lambda i: (i, 0)),
            pl.BlockSpec((k, n), lambda i: (0, 0)),
            pl.BlockSpec((1, n), lambda i: (0, 0)),
        ],
        out_specs=pl.BlockSpec((block_m, n), lambda i: (i, 0)),
        out_shape=jax.ShapeDtypeStruct((m, n), jnp.float32),
    )(x2d, w_all, b_all)


def _attn_body(rand_ref, q_ref,
               k0_ref, k1_ref, k2_ref, k3_ref, k4_ref, k5_ref,
               v0_ref, v1_ref, v2_ref, v3_ref, v4_ref, v5_ref,
               o_ref):
    i = pl.program_id(2)
    q = q_ref[0, 0] * SCALING

    k_refs = (k0_ref, k1_ref, k2_ref, k3_ref, k4_ref, k5_ref)
    scores = jnp.concatenate(
        [jnp.dot(q, kr[0, 0].T, preferred_element_type=jnp.float32)
         for kr in k_refs], axis=1)

    col = jax.lax.broadcasted_iota(jnp.int32, scores.shape, 1)
    scores = jnp.where((i == 0) & (col < BS), NEG_INF, scores)
    scores = jnp.where((i == NUM_BLOCKS - 1) & (col >= 2 * BS) & (col < 3 * BS),
                       NEG_INF, scores)

    p = jax.nn.softmax(scores, axis=-1)

    v_refs = (v0_ref, v1_ref, v2_ref, v3_ref, v4_ref, v5_ref)
    ctx = jnp.zeros((BS, HEAD_DIM), jnp.float32)
    for j, vr in enumerate(v_refs):
        ctx += jnp.dot(p[:, j * BS:(j + 1) * BS], vr[0, 0],
                       preferred_element_type=jnp.float32)
    o_ref[0, 0] = ctx


def _kv_spec(which):
    if which == 0:
        idx = lambda b, h, i, rand_ref: (b, h, jnp.maximum(i - 1, 0), 0)
    elif which == 1:
        idx = lambda b, h, i, rand_ref: (b, h, i, 0)
    elif which == 2:
        idx = lambda b, h, i, rand_ref: (b, h, jnp.minimum(i + 1, NUM_BLOCKS - 1), 0)
    else:
        r = which - 3
        idx = lambda b, h, i, rand_ref: (
            b, h, rand_ref[((b * NUM_HEADS + h) * NUM_BLOCKS + i) * N_RAND + r], 0)
    return pl.BlockSpec((1, 1, BS, HEAD_DIM), idx)


def _block_attn(q4, k4, v4, rand_attn):
    bsz = q4.shape[0]
    qspec = pl.BlockSpec((1, 1, BS, HEAD_DIM),
                         lambda b, h, i, rand_ref: (b, h, i, 0))
    grid_spec = pltpu.PrefetchScalarGridSpec(
        num_scalar_prefetch=1,
        grid=(bsz, NUM_HEADS, NUM_BLOCKS),
        in_specs=[qspec]
                 + [_kv_spec(j) for j in range(6)]
                 + [_kv_spec(j) for j in range(6)],
        out_specs=qspec,
    )
    return pl.pallas_call(
        _attn_body,
        grid_spec=grid_spec,
        out_shape=jax.ShapeDtypeStruct(q4.shape, jnp.float32),
        compiler_params=pltpu.CompilerParams(
            dimension_semantics=("parallel", "parallel", "arbitrary")),
    )(rand_attn.reshape(-1), q4, k4, k4, k4, k4, k4, k4, v4, v4, v4, v4, v4, v4)


@functools.partial(jax.jit, static_argnames=())
def kernel(hidden_states, rand_attn, Wq, bq, Wk, bk, Wv, bv):
    bsz, seqlen, embed = hidden_states.shape

    w_all = jnp.concatenate([Wq.T, Wk.T, Wv.T], axis=1)
    b_all = jnp.concatenate([bq, bk, bv]).reshape(1, 3 * embed)

    x2d = hidden_states.reshape(bsz * seqlen, embed)
    qkv = _qkv_proj(x2d, w_all, b_all)

    def t4s(x):
        return (x.reshape(bsz, seqlen, NUM_HEADS, HEAD_DIM)
                .transpose(0, 2, 1, 3))

    q4 = t4s(qkv[:, :embed])
    k4 = t4s(qkv[:, embed:2 * embed])
    v4 = t4s(qkv[:, 2 * embed:])

    ctx = _block_attn(q4, k4, v4, rand_attn.astype(jnp.int32))

    return (ctx.transpose(0, 2, 1, 3).reshape(bsz, seqlen, embed))


# speedup vs baseline: 1.6522x; 1.6522x over previous
import functools

import jax
import jax.numpy as jnp
from jax.experimental import pallas as pl
from jax.experimental.pallas import tpu as pltpu

EMBED = 1024
NUM_HEADS = 16
HEAD_DIM = EMBED // NUM_HEADS
NUM_BLOCKS = 64
BS = 64
N_RAND = 3
SCALING = HEAD_DIM ** (-0.5)
NEG_INF = -1e30


def _proj_body(x_ref, w_ref, b_ref, o_ref):
    acc = jnp.dot(x_ref[...], w_ref[...], preferred_element_type=jnp.float32)
    o_ref[...] = acc + b_ref[...]


def _qkv_proj(x2d, w_all, b_all, block_m=512):
    m = x2d.shape[0]
    n = w_all.shape[1]
    k = x2d.shape[1]
    return pl.pallas_call(
        _proj_body,
        grid=(m // block_m,),
        in_specs=[
            pl.BlockSpec((block_m, k), lambda i: (i, 0)),
            pl.BlockSpec((k, n), lambda i: (0, 0)),
            pl.BlockSpec((1, n), lambda i: (0, 0)),
        ],
        out_specs=pl.BlockSpec((block_m, n), lambda i: (i, 0)),
        out_shape=jax.ShapeDtypeStruct((m, n), jnp.float32),
    )(x2d, w_all, b_all)


def _attn_body(rand_ref, q_ref, k_ref, v_ref, o_ref):
    b = pl.program_id(0)
    h = pl.program_id(1)
    base = (b * NUM_HEADS + h) * NUM_BLOCKS * N_RAND

    def blk(i, carry):
        q = q_ref[0, 0, pl.ds(i * BS, BS), :] * SCALING
        ws = jnp.minimum(jnp.maximum(i - 1, 0), NUM_BLOCKS - 3) * BS
        r0 = rand_ref[base + i * N_RAND]
        r1 = rand_ref[base + i * N_RAND + 1]
        r2 = rand_ref[base + i * N_RAND + 2]

        k6 = jnp.concatenate([
            k_ref[0, 0, pl.ds(ws, 3 * BS), :],
            k_ref[0, 0, pl.ds(r0 * BS, BS), :],
            k_ref[0, 0, pl.ds(r1 * BS, BS), :],
            k_ref[0, 0, pl.ds(r2 * BS, BS), :],
        ], axis=0)
        scores = jax.lax.dot_general(
            q, k6, (((1,), (1,)), ((), ())),
            preferred_element_type=jnp.float32)

        col = jax.lax.broadcasted_iota(jnp.int32, scores.shape, 1)
        scores = jnp.where((i == 0) & (col >= 2 * BS) & (col < 3 * BS),
                           NEG_INF, scores)
        scores = jnp.where((i == NUM_BLOCKS - 1) & (col < BS), NEG_INF, scores)

        p = jax.nn.softmax(scores, axis=-1)

        v6 = jnp.concatenate([
            v_ref[0, 0, pl.ds(ws, 3 * BS), :],
            v_ref[0, 0, pl.ds(r0 * BS, BS), :],
            v_ref[0, 0, pl.ds(r1 * BS, BS), :],
            v_ref[0, 0, pl.ds(r2 * BS, BS), :],
        ], axis=0)
        ctx = jnp.dot(p, v6, preferred_element_type=jnp.float32)
        o_ref[0, 0, pl.ds(i * BS, BS), :] = ctx
        return carry

    jax.lax.fori_loop(0, NUM_BLOCKS, blk, 0)


def _block_attn(q4, k4, v4, rand_attn):
    bsz, _, seqlen, _ = q4.shape
    whole = pl.BlockSpec((1, 1, seqlen, HEAD_DIM), lambda b, h, rand_ref: (b, h, 0, 0))
    grid_spec = pltpu.PrefetchScalarGridSpec(
        num_scalar_prefetch=1,
        grid=(bsz, NUM_HEADS),
        in_specs=[whole, whole, whole],
        out_specs=whole,
    )
    return pl.pallas_call(
        _attn_body,
        grid_spec=grid_spec,
        out_shape=jax.ShapeDtypeStruct(q4.shape, jnp.float32),
        compiler_params=pltpu.CompilerParams(
            dimension_semantics=("arbitrary", "arbitrary")),
    )(rand_attn.reshape(-1), q4, k4, v4)


@functools.partial(jax.jit, static_argnames=())
def kernel(hidden_states, rand_attn, Wq, bq, Wk, bk, Wv, bv):
    bsz, seqlen, embed = hidden_states.shape

    w_all = jnp.concatenate([Wq.T, Wk.T, Wv.T], axis=1)
    b_all = jnp.concatenate([bq, bk, bv]).reshape(1, 3 * embed)

    x2d = hidden_states.reshape(bsz * seqlen, embed)
    qkv = _qkv_proj(x2d, w_all, b_all)

    def t4s(x):
        return (x.reshape(bsz, seqlen, NUM_HEADS, HEAD_DIM)
                .transpose(0, 2, 1, 3))

    q4 = t4s(qkv[:, :embed])
    k4 = t4s(qkv[:, embed:2 * embed])
    v4 = t4s(qkv[:, 2 * embed:])

    ctx = _block_attn(q4, k4, v4, rand_attn.astype(jnp.int32))

    return (ctx.transpose(0, 2, 1, 3).reshape(bsz, seqlen, embed))


# speedup vs baseline: 1.8799x; 1.1378x over previous
import jax
import jax.numpy as jnp
from jax.experimental import pallas as pl
from jax.experimental.pallas import tpu as pltpu

EMBED = 1024
NUM_HEADS = 16
HEAD_DIM = EMBED // NUM_HEADS
NUM_BLOCKS = 64
BS = 64
N_RAND = 3
SCALING = HEAD_DIM ** (-0.5)
NEG_INF = -1e30


def _proj_body(x_ref, w_ref, b_ref, o_ref):
    acc = jnp.dot(x_ref[...], w_ref[...], preferred_element_type=jnp.float32)
    o_ref[...] = acc + b_ref[...]


def _qkv_proj(x2d, w_all, b_all, block_m=512):
    m = x2d.shape[0]
    n = w_all.shape[1]
    k = x2d.shape[1]
    return pl.pallas_call(
        _proj_body,
        grid=(m // block_m,),
        in_specs=[
            pl.BlockSpec((block_m, k), lambda i: (i, 0)),
            pl.BlockSpec((k, n), lambda i: (0, 0)),
            pl.BlockSpec((1, n), lambda i: (0, 0)),
        ],
        out_specs=pl.BlockSpec((block_m, n), lambda i: (i, 0)),
        out_shape=jax.ShapeDtypeStruct((m, n), jnp.float32),
    )(x2d, w_all, b_all)


def _attn_body(rand_ref, q_ref, k_ref, v_ref, o_ref):
    b = pl.program_id(0)
    h = pl.program_id(1)
    base = (b * NUM_HEADS + h) * NUM_BLOCKS * N_RAND

    def blk(i, carry):
        q = q_ref[0, pl.ds(i * BS, BS), 0, 0, :] * SCALING
        ws = jnp.minimum(jnp.maximum(i - 1, 0), NUM_BLOCKS - 3) * BS
        r0 = rand_ref[base + i * N_RAND]
        r1 = rand_ref[base + i * N_RAND + 1]
        r2 = rand_ref[base + i * N_RAND + 2]

        k6 = jnp.concatenate([
            k_ref[0, pl.ds(ws, 3 * BS), 0, 0, :],
            k_ref[0, pl.ds(r0 * BS, BS), 0, 0, :],
            k_ref[0, pl.ds(r1 * BS, BS), 0, 0, :],
            k_ref[0, pl.ds(r2 * BS, BS), 0, 0, :],
        ], axis=0)
        scores = jax.lax.dot_general(
            q, k6, (((1,), (1,)), ((), ())),
            preferred_element_type=jnp.float32)

        col = jax.lax.broadcasted_iota(jnp.int32, scores.shape, 1)
        scores = jnp.where((i == 0) & (col >= 2 * BS) & (col < 3 * BS),
                           NEG_INF, scores)
        scores = jnp.where((i == NUM_BLOCKS - 1) & (col < BS), NEG_INF, scores)

        p = jax.nn.softmax(scores, axis=-1)

        v6 = jnp.concatenate([
            v_ref[0, pl.ds(ws, 3 * BS), 0, 0, :],
            v_ref[0, pl.ds(r0 * BS, BS), 0, 0, :],
            v_ref[0, pl.ds(r1 * BS, BS), 0, 0, :],
            v_ref[0, pl.ds(r2 * BS, BS), 0, 0, :],
        ], axis=0)
        ctx = jnp.dot(p, v6, preferred_element_type=jnp.float32)
        o_ref[0, pl.ds(i * BS, BS), 0, 0, :] = ctx
        return carry

    jax.lax.fori_loop(0, NUM_BLOCKS, blk, 0, unroll=4)


def _block_attn(qkv, rand_attn, bsz, seqlen):
    def col_spec(group):
        return pl.BlockSpec(
            (1, seqlen, 1, 1, HEAD_DIM),
            lambda b, h, rand_ref, g=group: (b, 0, g * NUM_HEADS + h, 0, 0))

    grid_spec = pltpu.PrefetchScalarGridSpec(
        num_scalar_prefetch=1,
        grid=(bsz, NUM_HEADS),
        in_specs=[col_spec(0), col_spec(1), col_spec(2)],
        out_specs=pl.BlockSpec((1, seqlen, 1, 1, HEAD_DIM),
                               lambda b, h, rand_ref: (b, 0, h, 0, 0)),
    )
    return pl.pallas_call(
        _attn_body,
        grid_spec=grid_spec,
        out_shape=jax.ShapeDtypeStruct((bsz, seqlen, NUM_HEADS, 1, HEAD_DIM),
                                       jnp.float32),
        compiler_params=pltpu.CompilerParams(
            dimension_semantics=("arbitrary", "arbitrary")),
    )(rand_attn.reshape(-1), qkv, qkv, qkv)


def kernel(hidden_states, rand_attn, Wq, bq, Wk, bk, Wv, bv):
    bsz, seqlen, embed = hidden_states.shape

    w_all = jnp.concatenate([Wq.T, Wk.T, Wv.T], axis=1)
    b_all = jnp.concatenate([bq, bk, bv]).reshape(1, 3 * embed)

    x2d = hidden_states.reshape(bsz * seqlen, embed)
    qkv = _qkv_proj(x2d, w_all, b_all)
    qkv = qkv.reshape(bsz, seqlen, 3 * NUM_HEADS, 1, HEAD_DIM)

    ctx = _block_attn(qkv, rand_attn.astype(jnp.int32), bsz, seqlen)
    return ctx.reshape(bsz, seqlen, embed)
